# Initial kernel scaffold; baseline (speedup 1.0000x reference)
import jax
import jax.numpy as jnp
from jax.experimental import pallas as pl


def kernel(x, wbounds, data, feat0, feat1, feat2):
    raise NotImplementedError("write your pallas kernel here")


# speedup vs baseline: 4.6966x; 4.6966x over previous
import functools

import jax
import jax.numpy as jnp
import numpy as np
from jax import lax
from jax.experimental import pallas as pl
from jax.experimental.pallas import tpu as pltpu
from jax.experimental.pallas import tpu_sc as plsc

EPS = 1e-06
NUM_LEVELS = 16
BASE_RESOLUTION = 16
DESIRED_RESOLUTION = 2048
LEVEL_DIM = 2
NUM_FRAMES = 100
FEAT_F = 64
RESO = 256
N_POINTS = 65536

_PRIME = 2 ** 19
def _isprime(n):
    i = 2
    while i * i <= n:
        if n % i == 0:
            return False
        i += 1
    return True
while not _isprime(_PRIME):
    _PRIME += 1
PRIME = _PRIME

B_SCALE = (DESIRED_RESOLUTION / BASE_RESOLUTION) ** (1.0 / (NUM_LEVELS - 1))
SCALES = []
OFFSETS = [0]
START_HASH = -1
for _i in range(NUM_LEVELS):
    _res = int(BASE_RESOLUTION * B_SCALE ** _i)
    SCALES.append(_res)
    _n_e = (_res + 1) ** 3
    if _n_e > PRIME:
        if START_HASH < 0:
            START_HASH = _i
        _n_e = PRIME
    OFFSETS.append(OFFSETS[-1] + _n_e)
TOTAL_ENTRIES = OFFSETS[-1]

C2_HI, C2_LO = 19349663 >> 16, 19349663 & 0xFFFF
C3_HI, C3_LO = 83492791 >> 16, 83492791 & 0xFFFF
INV_P = np.float32(1.0 / PRIME)

NW = 32
PW = N_POINTS // NW
CHUNK = 16
NCHUNK = PW // CHUNK
CH = 3 * FEAT_F
TBL_ROWS = 3 * NUM_FRAMES * RESO


def _splat_i32(v):
    return jnp.full((16,), v, dtype=jnp.int32)


def _mod_p(v):
    q = (v.astype(jnp.float32) * INV_P).astype(jnp.int32)
    r = v - q * PRIME
    r = jnp.where(r < 0, r + PRIME, r)
    r = jnp.where(r >= PRIME, r - PRIME, r)
    return r


def _sc_body(tbl, data, posw, out, pos_v, dbuf, idxbuf, lobuf, ghash, outc,
             sem_d, sem_h):
    nc = 2
    wid = lax.axis_index("s") * nc + lax.axis_index("c")
    base_pt = wid * PW

    pltpu.sync_copy(posw.at[wid], pos_v)

    lane = lax.iota(jnp.int32, 16)
    onef = jnp.full((16,), 1.0, dtype=jnp.float32)

    def chunk_body(ci, _):
        s = ci.astype(jnp.int32) * CHUNK
        px = pos_v[0, pl.ds(s, CHUNK)]
        py = pos_v[1, pl.ds(s, CHUNK)]
        pz = pos_v[2, pl.ds(s, CHUNK)]
        pt = pos_v[3, pl.ds(s, CHUNK)]

        tint = pt.astype(jnp.int32)
        rowbase = tint * RESO

        wxs = []
        descs = []
        for j, pj in enumerate((px, py, pz)):
            g = 2.0 * pj - 1.0
            gx = ((g + 1.0) * 0.5) * np.float32(RESO - 1)
            x0 = gx.astype(jnp.int32)
            wxs.append(gx - x0.astype(jnp.float32))
            x1 = jnp.minimum(x0 + 1, RESO - 1)
            r0 = j * (NUM_FRAMES * RESO) + rowbase + x0
            r1 = j * (NUM_FRAMES * RESO) + rowbase + x1
            descs.append(pltpu.async_copy(tbl.at[r0], dbuf.at[2 * j], sem_d))
            descs.append(pltpu.async_copy(tbl.at[r1], dbuf.at[2 * j + 1], sem_d))
        for d in descs:
            d.wait()

        def ch_body(c, accs):
            new = []
            for i in range(3):
                chn = _splat_i32(i * FEAT_F) + c.astype(jnp.int32)
                prod = onef
                for j in range(3):
                    v0 = plsc.load_gather(dbuf, [_splat_i32(2 * j), lane, chn])
                    v1 = plsc.load_gather(dbuf, [_splat_i32(2 * j + 1), lane, chn])
                    prod = prod * (v0 + wxs[j] * (v1 - v0))
                new.append(accs[i] + prod)
            return tuple(new)

        zero = jnp.zeros((16,), dtype=jnp.float32)
        d0, d1, d2 = lax.fori_loop(0, FEAT_F, ch_body, (zero, zero, zero))

        hx = jnp.clip(px + d0, 0.0, np.float32(1.0 - EPS))
        hy = jnp.clip(py + d1, 0.0, np.float32(1.0 - EPS))
        hz = jnp.clip(pz + d2, 0.0, np.float32(1.0 - EPS))

        level_w = []
        hdescs = []
        for l in range(NUM_LEVELS):
            sc = np.float32(SCALES[l])
            fx, fy, fz = hx * sc, hy * sc, hz * sc
            ix = fx.astype(jnp.int32)
            iy = fy.astype(jnp.int32)
            iz = fz.astype(jnp.int32)
            ox = fx - ix.astype(jnp.float32)
            oy = fy - iy.astype(jnp.float32)
            oz = fz - iz.astype(jnp.float32)
            off = OFFSETS[l]
            def put(idx, l, c):
                idxbuf[l, pl.ds(c * 16, 16)] = lax.shift_right_logical(idx, 2)
                lobuf[l, pl.ds(c * 16, 16)] = jnp.bitwise_and(idx, 3) * 2

            if l < START_HASH:
                sp = SCALES[l] + 1
                b0 = ix * (sp * sp) + iy * sp + iz + off
                for c in range(8):
                    bx, by, bz = c >> 2, (c >> 1) & 1, c & 1
                    idx = b0 + (bx * sp * sp + by * sp + bz)
                    put(idx, l, c)
            else:
                h2, l2, h3, l3 = [], [], [], []
                for b in (iy, iy + 1):
                    lo = b * C2_LO
                    h2.append(b * C2_HI + (lo >> 16))
                    l2.append(jnp.bitwise_and(lo, 0xFFFF))
                for b in (iz, iz + 1):
                    lo = b * C3_LO
                    h3.append(b * C3_HI + (lo >> 16))
                    l3.append(jnp.bitwise_and(lo, 0xFFFF))
                wmod = {}
                for by in range(2):
                    for bz in range(2):
                        hh = jnp.bitwise_xor(h2[by], h3[bz])
                        wmod[(by, bz)] = _mod_p(_mod_p(hh) * 256)
                for c in range(8):
                    bx, by, bz = c >> 2, (c >> 1) & 1, c & 1
                    ll = jnp.bitwise_xor(jnp.bitwise_xor(ix + bx, l2[by]),
                                         l3[bz])
                    idx = _mod_p(wmod[(by, bz)] * 256 + ll) + off
                    put(idx, l, c)
            hdescs.append(
                pltpu.async_copy(data.at[idxbuf.at[l]], ghash.at[l], sem_h))
            ax = (onef - ox, ox)
            ay = (onef - oy, oy)
            az = (onef - oz, oz)
            level_w.append([ax[c >> 2] * ay[(c >> 1) & 1] * az[c & 1]
                            for c in range(8)])

        for l in range(NUM_LEVELS):
            hdescs[l].wait()
            lovs = [plsc.load_gather(lobuf, [_splat_i32(l),
                                             _splat_i32(c * 16) + lane])
                    for c in range(8)]
            for chn in range(LEVEL_DIM):
                val = jnp.zeros((16,), dtype=jnp.float32)
                for c in range(8):
                    gv = plsc.load_gather(
                        ghash, [_splat_i32(l), _splat_i32(c * 16) + lane,
                                lovs[c] + chn])
                    val = val + level_w[l][c] * gv
                plsc.store_scatter(outc, [lane, _splat_i32(l * LEVEL_DIM + chn)],
                                   val)

        pltpu.sync_copy(outc, out.at[pl.ds(base_pt + s, CHUNK)])
        return _

    lax.fori_loop(0, NCHUNK, chunk_body, None)


@jax.jit
def _run(tbl, data, posw):
    mesh = plsc.VectorSubcoreMesh(core_axis_name="c", subcore_axis_name="s")
    return pl.kernel(
        _sc_body,
        out_type=jax.ShapeDtypeStruct((N_POINTS, NUM_LEVELS * LEVEL_DIM),
                                      jnp.float32),
        mesh=mesh,
        compiler_params=pltpu.CompilerParams(use_tc_tiling_on_sc=False,
                                             needs_layout_passes=False),
        scratch_types=[
            pltpu.VMEM((4, PW), jnp.float32),
            pltpu.VMEM((6, CHUNK, CH), jnp.float32),
            pltpu.VMEM((NUM_LEVELS, 8 * CHUNK), jnp.int32),
            pltpu.VMEM((NUM_LEVELS, 8 * CHUNK), jnp.int32),
            pltpu.VMEM((NUM_LEVELS, 8 * CHUNK, 8), jnp.float32),
            pltpu.VMEM((CHUNK, NUM_LEVELS * LEVEL_DIM), jnp.float32),
            pltpu.SemaphoreType.DMA,
            pltpu.SemaphoreType.DMA,
        ],
    )(tbl, data, posw)


def kernel(x, wbounds, data, feat0, feat1, feat2):
    with jax.enable_x64(False):
        return _kernel32(x, wbounds, data, feat0, feat1, feat2)


def _kernel32(x, wbounds, data, feat0, feat1, feat2):
    xyz = x[:, :3].astype(jnp.float32)
    t = x[:, 3].astype(jnp.float32)
    lo = wbounds[:3].astype(jnp.float32)
    hi = wbounds[3:6].astype(jnp.float32)
    inp = (jnp.clip(xyz, lo, hi) - lo) / jnp.max(hi - lo)
    posw = jnp.concatenate([inp, t[:, None]], axis=1)
    posw = posw.T.reshape(4, NW, PW).transpose(1, 0, 2)
    feats = jnp.stack([feat0, feat1, feat2], axis=0)
    tbl = jnp.transpose(feats, (1, 3, 4, 0, 2)).reshape(TBL_ROWS, CH)
    data8 = data.astype(jnp.float32).reshape(TOTAL_ENTRIES * LEVEL_DIM // 8, 8)
    return _run(tbl, data8, posw.astype(jnp.float32))


# speedup vs baseline: 4.7491x; 1.0112x over previous
import functools

import jax
import jax.numpy as jnp
import numpy as np
from jax import lax
from jax.experimental import pallas as pl
from jax.experimental.pallas import tpu as pltpu
from jax.experimental.pallas import tpu_sc as plsc

EPS = 1e-06
NUM_LEVELS = 16
BASE_RESOLUTION = 16
DESIRED_RESOLUTION = 2048
LEVEL_DIM = 2
NUM_FRAMES = 100
FEAT_F = 64
RESO = 256
N_POINTS = 65536

_PRIME = 2 ** 19
def _isprime(n):
    i = 2
    while i * i <= n:
        if n % i == 0:
            return False
        i += 1
    return True
while not _isprime(_PRIME):
    _PRIME += 1
PRIME = _PRIME

B_SCALE = (DESIRED_RESOLUTION / BASE_RESOLUTION) ** (1.0 / (NUM_LEVELS - 1))
SCALES = []
OFFSETS = [0]
START_HASH = -1
for _i in range(NUM_LEVELS):
    _res = int(BASE_RESOLUTION * B_SCALE ** _i)
    SCALES.append(_res)
    _n_e = (_res + 1) ** 3
    if _n_e > PRIME:
        if START_HASH < 0:
            START_HASH = _i
        _n_e = PRIME
    OFFSETS.append(OFFSETS[-1] + _n_e)
TOTAL_ENTRIES = OFFSETS[-1]

C2_HI, C2_LO = 19349663 >> 16, 19349663 & 0xFFFF
C3_HI, C3_LO = 83492791 >> 16, 83492791 & 0xFFFF
INV_P = np.float32(1.0 / PRIME)

NW = 32
PW = N_POINTS // NW
CHUNK = 16
NCHUNK = PW // CHUNK
CH = 3 * FEAT_F
TBL_ROWS = 3 * NUM_FRAMES * RESO


def _splat_i32(v):
    return jnp.full((16,), v, dtype=jnp.int32)


def _mod_p(v):
    q = (v.astype(jnp.float32) * INV_P).astype(jnp.int32)
    r = v - q * PRIME
    r = jnp.where(r < 0, r + PRIME, r)
    r = jnp.where(r >= PRIME, r - PRIME, r)
    return r


def _sc_body(tbl, data, posw, out, pos_v, dbuf, idxbuf, lobuf, ghash, outc,
             sem_d, sem_h):
    nc = 2
    wid = lax.axis_index("s") * nc + lax.axis_index("c")
    base_pt = wid * PW

    pltpu.sync_copy(posw.at[wid], pos_v)

    lane = lax.iota(jnp.int32, 16)
    onef = jnp.full((16,), 1.0, dtype=jnp.float32)

    def chunk_body(ci, _):
        s = ci.astype(jnp.int32) * CHUNK
        px = pos_v[0, pl.ds(s, CHUNK)]
        py = pos_v[1, pl.ds(s, CHUNK)]
        pz = pos_v[2, pl.ds(s, CHUNK)]
        pt = pos_v[3, pl.ds(s, CHUNK)]

        tint = pt.astype(jnp.int32)
        rowbase = tint * RESO

        wxs = []
        descs = []
        for j, pj in enumerate((px, py, pz)):
            g = 2.0 * pj - 1.0
            gx = ((g + 1.0) * 0.5) * np.float32(RESO - 1)
            x0 = gx.astype(jnp.int32)
            wxs.append(gx - x0.astype(jnp.float32))
            x1 = jnp.minimum(x0 + 1, RESO - 1)
            r0 = j * (NUM_FRAMES * RESO) + rowbase + x0
            r1 = j * (NUM_FRAMES * RESO) + rowbase + x1
            descs.append(pltpu.async_copy(tbl.at[r0], dbuf.at[2 * j], sem_d))
            descs.append(pltpu.async_copy(tbl.at[r1], dbuf.at[2 * j + 1], sem_d))
        for d in descs:
            d.wait()

        def ch_body(c, accs):
            new = []
            for i in range(3):
                chn = _splat_i32(i * FEAT_F) + c.astype(jnp.int32)
                prod = onef
                for j in range(3):
                    v0 = plsc.load_gather(dbuf, [_splat_i32(2 * j), lane, chn])
                    v1 = plsc.load_gather(dbuf, [_splat_i32(2 * j + 1), lane, chn])
                    prod = prod * (v0 + wxs[j] * (v1 - v0))
                new.append(accs[i] + prod)
            return tuple(new)

        zero = jnp.zeros((16,), dtype=jnp.float32)
        d0, d1, d2 = lax.fori_loop(0, FEAT_F, ch_body, (zero, zero, zero))

        hx = jnp.clip(px + d0, 0.0, np.float32(1.0 - EPS))
        hy = jnp.clip(py + d1, 0.0, np.float32(1.0 - EPS))
        hz = jnp.clip(pz + d2, 0.0, np.float32(1.0 - EPS))

        level_w = []
        hdescs = []
        for l in range(NUM_LEVELS):
            sc = np.float32(SCALES[l])
            fx, fy, fz = hx * sc, hy * sc, hz * sc
            ix = fx.astype(jnp.int32)
            iy = fy.astype(jnp.int32)
            iz = fz.astype(jnp.int32)
            ox = fx - ix.astype(jnp.float32)
            oy = fy - iy.astype(jnp.float32)
            oz = fz - iz.astype(jnp.float32)
            off = OFFSETS[l]
            def put(idx, l, c):
                idxbuf[l, pl.ds(c * 16, 16)] = lax.shift_right_logical(idx, 2)
                lobuf[l, pl.ds(c * 16, 16)] = jnp.bitwise_and(idx, 3) * 2

            if l < START_HASH:
                sp = SCALES[l] + 1
                b0 = ix * (sp * sp) + iy * sp + iz + off
                for c in range(8):
                    bx, by, bz = c >> 2, (c >> 1) & 1, c & 1
                    idx = b0 + (bx * sp * sp + by * sp + bz)
                    put(idx, l, c)
            else:
                h2, l2, h3, l3 = [], [], [], []
                for b in (iy, iy + 1):
                    lo = b * C2_LO
                    h2.append(b * C2_HI + (lo >> 16))
                    l2.append(jnp.bitwise_and(lo, 0xFFFF))
                for b in (iz, iz + 1):
                    lo = b * C3_LO
                    h3.append(b * C3_HI + (lo >> 16))
                    l3.append(jnp.bitwise_and(lo, 0xFFFF))
                wmod = {}
                for by in range(2):
                    for bz in range(2):
                        hh = jnp.bitwise_xor(h2[by], h3[bz])
                        wmod[(by, bz)] = _mod_p(_mod_p(hh) * 256)
                for c in range(8):
                    bx, by, bz = c >> 2, (c >> 1) & 1, c & 1
                    ll = jnp.bitwise_xor(jnp.bitwise_xor(ix + bx, l2[by]),
                                         l3[bz])
                    idx = _mod_p(wmod[(by, bz)] * 256 + ll) + off
                    put(idx, l, c)
            hdescs.append(
                pltpu.async_copy(data.at[idxbuf.at[l]], ghash.at[l], sem_h))
            ax = (onef - ox, ox)
            ay = (onef - oy, oy)
            az = (onef - oz, oz)
            level_w.append([ax[c >> 2] * ay[(c >> 1) & 1] * az[c & 1]
                            for c in range(8)])

        for l in range(NUM_LEVELS):
            hdescs[l].wait()
            lovs = [plsc.load_gather(lobuf, [_splat_i32(l),
                                             _splat_i32(c * 16) + lane])
                    for c in range(8)]
            for chn in range(LEVEL_DIM):
                val = jnp.zeros((16,), dtype=jnp.float32)
                for c in range(8):
                    gv = plsc.load_gather(
                        ghash, [_splat_i32(l), _splat_i32(c * 16) + lane,
                                lovs[c] + chn])
                    val = val + level_w[l][c] * gv
                plsc.store_scatter(outc, [lane, _splat_i32(l * LEVEL_DIM + chn)],
                                   val)

        pltpu.sync_copy(outc, out.at[pl.ds(base_pt + s, CHUNK)])
        return _

    lax.fori_loop(0, NCHUNK, chunk_body, None)


@jax.jit
def _run(tbl, data, posw):
    mesh = plsc.VectorSubcoreMesh(core_axis_name="c", subcore_axis_name="s")
    return pl.kernel(
        _sc_body,
        out_type=jax.ShapeDtypeStruct((N_POINTS, NUM_LEVELS * LEVEL_DIM),
                                      jnp.float32),
        mesh=mesh,
        compiler_params=pltpu.CompilerParams(use_tc_tiling_on_sc=False,
                                             needs_layout_passes=False),
        scratch_types=[
            pltpu.VMEM((4, PW), jnp.float32),
            pltpu.VMEM((6, CHUNK, CH), jnp.float32),
            pltpu.VMEM((NUM_LEVELS, 8 * CHUNK), jnp.int32),
            pltpu.VMEM((NUM_LEVELS, 8 * CHUNK), jnp.int32),
            pltpu.VMEM((NUM_LEVELS, 8 * CHUNK, 8), jnp.float32),
            pltpu.VMEM((CHUNK, NUM_LEVELS * LEVEL_DIM), jnp.float32),
            pltpu.SemaphoreType.DMA,
            pltpu.SemaphoreType.DMA,
        ],
    )(tbl, data, posw)


def kernel(x, wbounds, data, feat0, feat1, feat2):
    with jax.enable_x64(False):
        return _kernel32(x, wbounds, data, feat0, feat1, feat2)


_TF = 10


def _tbl_body(f0, f1, f2, out):
    for tt in range(_TF):
        cols = [f[0, :, pl.ds(tt * RESO, RESO)].T for f in (f0, f1, f2)]
        out[0, 0, tt] = jnp.concatenate(cols, axis=1)


def _build_tbl(feat0, feat1, feat2):
    nt = NUM_FRAMES // _TF
    spec = pl.BlockSpec((1, FEAT_F, _TF * RESO), lambda j, t: (j, 0, t))
    out = pl.pallas_call(
        _tbl_body,
        grid=(3, nt),
        in_specs=[spec, spec, spec],
        out_specs=pl.BlockSpec((1, 1, _TF, RESO, CH),
                               lambda j, t: (j, t, 0, 0, 0)),
        out_shape=jax.ShapeDtypeStruct((3, nt, _TF, RESO, CH), jnp.float32),
    )(feat0.reshape(3, FEAT_F, NUM_FRAMES * RESO),
      feat1.reshape(3, FEAT_F, NUM_FRAMES * RESO),
      feat2.reshape(3, FEAT_F, NUM_FRAMES * RESO))
    return out.reshape(TBL_ROWS, CH)


def _kernel32(x, wbounds, data, feat0, feat1, feat2):
    xyz = x[:, :3].astype(jnp.float32)
    t = x[:, 3].astype(jnp.float32)
    lo = wbounds[:3].astype(jnp.float32)
    hi = wbounds[3:6].astype(jnp.float32)
    inp = (jnp.clip(xyz, lo, hi) - lo) / jnp.max(hi - lo)
    posw = jnp.concatenate([inp, t[:, None]], axis=1)
    posw = posw.T.reshape(4, NW, PW).transpose(1, 0, 2)
    tbl = _build_tbl(feat0.astype(jnp.float32), feat1.astype(jnp.float32),
                     feat2.astype(jnp.float32))
    data8 = data.astype(jnp.float32).reshape(TOTAL_ENTRIES * LEVEL_DIM // 8, 8)
    return _run(tbl, data8, posw.astype(jnp.float32))


# speedup vs baseline: 13.6980x; 2.8843x over previous
import functools

import jax
import jax.numpy as jnp
import numpy as np
from jax import lax
from jax.experimental import pallas as pl
from jax.experimental.pallas import tpu as pltpu
from jax.experimental.pallas import tpu_sc as plsc

EPS = 1e-06
NUM_LEVELS = 16
BASE_RESOLUTION = 16
DESIRED_RESOLUTION = 2048
LEVEL_DIM = 2
NUM_FRAMES = 100
FEAT_F = 64
RESO = 256
N_POINTS = 65536

_PRIME = 2 ** 19
def _isprime(n):
    i = 2
    while i * i <= n:
        if n % i == 0:
            return False
        i += 1
    return True
while not _isprime(_PRIME):
    _PRIME += 1
PRIME = _PRIME

B_SCALE = (DESIRED_RESOLUTION / BASE_RESOLUTION) ** (1.0 / (NUM_LEVELS - 1))
SCALES = []
OFFSETS = [0]
START_HASH = -1
for _i in range(NUM_LEVELS):
    _res = int(BASE_RESOLUTION * B_SCALE ** _i)
    SCALES.append(_res)
    _n_e = (_res + 1) ** 3
    if _n_e > PRIME:
        if START_HASH < 0:
            START_HASH = _i
        _n_e = PRIME
    OFFSETS.append(OFFSETS[-1] + _n_e)
TOTAL_ENTRIES = OFFSETS[-1]

C2_HI, C2_LO = 19349663 >> 16, 19349663 & 0xFFFF
C3_HI, C3_LO = 83492791 >> 16, 83492791 & 0xFFFF
INV_P = np.float32(1.0 / PRIME)

NW = 32
PW = N_POINTS // NW
CHUNK = 16
NCHUNK = PW // CHUNK
CH = 3 * FEAT_F
TBL_ROWS = 3 * NUM_FRAMES * RESO


def _splat_i32(v):
    return jnp.full((16,), v, dtype=jnp.int32)


def _mod_p(v):
    q = (v.astype(jnp.float32) * INV_P).astype(jnp.int32)
    r = v - q * PRIME
    r = jnp.where(r < 0, r + PRIME, r)
    r = jnp.where(r >= PRIME, r - PRIME, r)
    return r


def _sc_body(tbl, data, posw, out, pos_v, dbuf, idxbuf, ghash, outc,
             sem_d, sem_h):
    nc = 2
    wid = lax.axis_index("s") * nc + lax.axis_index("c")
    base_pt = wid * PW

    pltpu.sync_copy(posw.at[wid], pos_v)

    lane = lax.iota(jnp.int32, 16)
    onef = jnp.full((16,), 1.0, dtype=jnp.float32)

    def chunk_body(ci, _):
        s = ci.astype(jnp.int32) * CHUNK
        px = pos_v[0, pl.ds(s, CHUNK)]
        py = pos_v[1, pl.ds(s, CHUNK)]
        pz = pos_v[2, pl.ds(s, CHUNK)]
        pt = pos_v[3, pl.ds(s, CHUNK)]

        tint = pt.astype(jnp.int32)
        rowbase = tint * RESO

        wxs = []
        descs = []
        for j, pj in enumerate((px, py, pz)):
            g = 2.0 * pj - 1.0
            gx = ((g + 1.0) * 0.5) * np.float32(RESO - 1)
            x0 = gx.astype(jnp.int32)
            wxs.append(gx - x0.astype(jnp.float32))
            x1 = jnp.minimum(x0 + 1, RESO - 1)
            r0 = j * (NUM_FRAMES * RESO) + rowbase + x0
            r1 = j * (NUM_FRAMES * RESO) + rowbase + x1
            descs.append(pltpu.async_copy(tbl.at[r0], dbuf.at[2 * j], sem_d))
            descs.append(pltpu.async_copy(tbl.at[r1], dbuf.at[2 * j + 1], sem_d))
        for d in descs:
            d.wait()

        def ch_body(c, accs):
            new = []
            for i in range(3):
                chn = _splat_i32(i * FEAT_F) + c.astype(jnp.int32)
                prod = onef
                for j in range(3):
                    v0 = plsc.load_gather(dbuf, [_splat_i32(2 * j), lane, chn])
                    v1 = plsc.load_gather(dbuf, [_splat_i32(2 * j + 1), lane, chn])
                    prod = prod * (v0 + wxs[j] * (v1 - v0))
                new.append(accs[i] + prod)
            return tuple(new)

        zero = jnp.zeros((16,), dtype=jnp.float32)
        d0, d1, d2 = lax.fori_loop(0, FEAT_F, ch_body, (zero, zero, zero))

        hx = jnp.clip(px + d0, 0.0, np.float32(1.0 - EPS))
        hy = jnp.clip(py + d1, 0.0, np.float32(1.0 - EPS))
        hz = jnp.clip(pz + d2, 0.0, np.float32(1.0 - EPS))

        level_w = []
        hdescs = []
        for l in range(NUM_LEVELS):
            sc = np.float32(SCALES[l])
            fx, fy, fz = hx * sc, hy * sc, hz * sc
            ix = fx.astype(jnp.int32)
            iy = fy.astype(jnp.int32)
            iz = fz.astype(jnp.int32)
            ox = fx - ix.astype(jnp.float32)
            oy = fy - iy.astype(jnp.float32)
            oz = fz - iz.astype(jnp.float32)
            off = OFFSETS[l]
            def put(idx, l, c):
                idxbuf[l, 0, pl.ds(c * 16, 16)] = idx
                idxbuf[l, 1, pl.ds(c * 16, 16)] = idx + TOTAL_ENTRIES

            if l < START_HASH:
                sp = SCALES[l] + 1
                b0 = ix * (sp * sp) + iy * sp + iz + off
                for c in range(8):
                    bx, by, bz = c >> 2, (c >> 1) & 1, c & 1
                    idx = b0 + (bx * sp * sp + by * sp + bz)
                    put(idx, l, c)
            else:
                h2, l2, h3, l3 = [], [], [], []
                for b in (iy, iy + 1):
                    lo = b * C2_LO
                    h2.append(b * C2_HI + (lo >> 16))
                    l2.append(jnp.bitwise_and(lo, 0xFFFF))
                for b in (iz, iz + 1):
                    lo = b * C3_LO
                    h3.append(b * C3_HI + (lo >> 16))
                    l3.append(jnp.bitwise_and(lo, 0xFFFF))
                wmod = {}
                for by in range(2):
                    for bz in range(2):
                        hh = jnp.bitwise_xor(h2[by], h3[bz])
                        wmod[(by, bz)] = _mod_p(_mod_p(hh) * 256)
                for c in range(8):
                    bx, by, bz = c >> 2, (c >> 1) & 1, c & 1
                    ll = jnp.bitwise_xor(jnp.bitwise_xor(ix + bx, l2[by]),
                                         l3[bz])
                    idx = _mod_p(wmod[(by, bz)] * 256 + ll) + off
                    put(idx, l, c)
            hdescs.append(
                pltpu.async_copy(data.at[idxbuf.at[l, 0]], ghash.at[l, 0],
                                 sem_h))
            hdescs.append(
                pltpu.async_copy(data.at[idxbuf.at[l, 1]], ghash.at[l, 1],
                                 sem_h))
            ax = (onef - ox, ox)
            ay = (onef - oy, oy)
            az = (onef - oz, oz)
            level_w.append([ax[c >> 2] * ay[(c >> 1) & 1] * az[c & 1]
                            for c in range(8)])

        for l in range(NUM_LEVELS):
            hdescs[2 * l].wait()
            hdescs[2 * l + 1].wait()
            for chn in range(LEVEL_DIM):
                val = jnp.zeros((16,), dtype=jnp.float32)
                for c in range(8):
                    gv = plsc.load_gather(
                        ghash, [_splat_i32(l), _splat_i32(chn),
                                _splat_i32(c * 16) + lane])
                    val = val + level_w[l][c] * gv
                plsc.store_scatter(outc, [lane, _splat_i32(l * LEVEL_DIM + chn)],
                                   val)

        pltpu.sync_copy(outc, out.at[pl.ds(base_pt + s, CHUNK)])
        return _

    lax.fori_loop(0, NCHUNK, chunk_body, None)


@jax.jit
def _run(tbl, data, posw):
    mesh = plsc.VectorSubcoreMesh(core_axis_name="c", subcore_axis_name="s")
    return pl.kernel(
        _sc_body,
        out_type=jax.ShapeDtypeStruct((N_POINTS, NUM_LEVELS * LEVEL_DIM),
                                      jnp.float32),
        mesh=mesh,
        compiler_params=pltpu.CompilerParams(use_tc_tiling_on_sc=False,
                                             needs_layout_passes=False),
        scratch_types=[
            pltpu.VMEM((4, PW), jnp.float32),
            pltpu.VMEM((6, CHUNK, CH), jnp.float32),
            pltpu.VMEM((NUM_LEVELS, LEVEL_DIM, 8 * CHUNK), jnp.int32),
            pltpu.VMEM((NUM_LEVELS, LEVEL_DIM, 8 * CHUNK), jnp.float32),
            pltpu.VMEM((CHUNK, NUM_LEVELS * LEVEL_DIM), jnp.float32),
            pltpu.SemaphoreType.DMA,
            pltpu.SemaphoreType.DMA,
        ],
    )(tbl, data, posw)


def kernel(x, wbounds, data, feat0, feat1, feat2):
    with jax.enable_x64(False):
        return _kernel32(x, wbounds, data, feat0, feat1, feat2)


_TF = 10


def _tbl_body(f0, f1, f2, out):
    for tt in range(_TF):
        cols = [f[0, :, pl.ds(tt * RESO, RESO)].T for f in (f0, f1, f2)]
        out[0, 0, tt] = jnp.concatenate(cols, axis=1)


def _build_tbl(feat0, feat1, feat2):
    nt = NUM_FRAMES // _TF
    spec = pl.BlockSpec((1, FEAT_F, _TF * RESO), lambda j, t: (j, 0, t))
    out = pl.pallas_call(
        _tbl_body,
        grid=(3, nt),
        in_specs=[spec, spec, spec],
        out_specs=pl.BlockSpec((1, 1, _TF, RESO, CH),
                               lambda j, t: (j, t, 0, 0, 0)),
        out_shape=jax.ShapeDtypeStruct((3, nt, _TF, RESO, CH), jnp.float32),
    )(feat0.reshape(3, FEAT_F, NUM_FRAMES * RESO),
      feat1.reshape(3, FEAT_F, NUM_FRAMES * RESO),
      feat2.reshape(3, FEAT_F, NUM_FRAMES * RESO))
    return out.reshape(TBL_ROWS, CH)


def _kernel32(x, wbounds, data, feat0, feat1, feat2):
    xyz = x[:, :3].astype(jnp.float32)
    t = x[:, 3].astype(jnp.float32)
    lo = wbounds[:3].astype(jnp.float32)
    hi = wbounds[3:6].astype(jnp.float32)
    inp = (jnp.clip(xyz, lo, hi) - lo) / jnp.max(hi - lo)
    posw = jnp.concatenate([inp, t[:, None]], axis=1)
    posw = posw.T.reshape(4, NW, PW).transpose(1, 0, 2)
    tbl = _build_tbl(feat0.astype(jnp.float32), feat1.astype(jnp.float32),
                     feat2.astype(jnp.float32))
    data_flat = jnp.transpose(data.astype(jnp.float32)).ravel()
    return _run(tbl, data_flat, posw.astype(jnp.float32))


# speedup vs baseline: 17.3857x; 1.2692x over previous
import functools

import jax
import jax.numpy as jnp
import numpy as np
from jax import lax
from jax.experimental import pallas as pl
from jax.experimental.pallas import tpu as pltpu
from jax.experimental.pallas import tpu_sc as plsc

EPS = 1e-06
NUM_LEVELS = 16
BASE_RESOLUTION = 16
DESIRED_RESOLUTION = 2048
LEVEL_DIM = 2
NUM_FRAMES = 100
FEAT_F = 64
RESO = 256
N_POINTS = 65536

_PRIME = 2 ** 19
def _isprime(n):
    i = 2
    while i * i <= n:
        if n % i == 0:
            return False
        i += 1
    return True
while not _isprime(_PRIME):
    _PRIME += 1
PRIME = _PRIME

B_SCALE = (DESIRED_RESOLUTION / BASE_RESOLUTION) ** (1.0 / (NUM_LEVELS - 1))
SCALES = []
OFFSETS = [0]
START_HASH = -1
for _i in range(NUM_LEVELS):
    _res = int(BASE_RESOLUTION * B_SCALE ** _i)
    SCALES.append(_res)
    _n_e = (_res + 1) ** 3
    if _n_e > PRIME:
        if START_HASH < 0:
            START_HASH = _i
        _n_e = PRIME
    OFFSETS.append(OFFSETS[-1] + _n_e)
TOTAL_ENTRIES = OFFSETS[-1]

C2_HI, C2_LO = 19349663 >> 16, 19349663 & 0xFFFF
C3_HI, C3_LO = 83492791 >> 16, 83492791 & 0xFFFF
INV_P = np.float32(1.0 / PRIME)

NW = 32
PW = N_POINTS // NW
CHUNK = 16
NCHUNK = PW // CHUNK
CH = 3 * FEAT_F
TBL_ROWS = 3 * NUM_FRAMES * RESO


def _splat_i32(v):
    return jnp.full((16,), v, dtype=jnp.int32)


def _mod_p(v):
    q = (v.astype(jnp.float32) * INV_P).astype(jnp.int32)
    r = v - q * PRIME
    r = jnp.where(r < 0, r + PRIME, r)
    r = jnp.where(r >= PRIME, r - PRIME, r)
    return r


def _sc_body(tbl, data, posw, out, pos_v, dbuf, idxbuf, ghash, posq, outc,
             sem_d, sem_h, sem_o):
    nc = 2
    wid = lax.axis_index("s") * nc + lax.axis_index("c")
    base_pt = wid * PW

    pltpu.sync_copy(posw.at[wid], pos_v)

    lane = lax.iota(jnp.int32, 16)
    onef = jnp.full((16,), 1.0, dtype=jnp.float32)

    def chunk_cols(k):
        s = k * CHUNK
        return [pos_v[d, pl.ds(s, CHUNK)] for d in range(4)]

    def xweights(pj):
        g = 2.0 * pj - 1.0
        gx = ((g + 1.0) * 0.5) * np.float32(RESO - 1)
        x0 = gx.astype(jnp.int32)
        return x0, gx - x0.astype(jnp.float32)

    def fire_delta(k, slot3):
        px, py, pz, pt = chunk_cols(k)
        rowbase = pt.astype(jnp.int32) * RESO
        for j, pj in enumerate((px, py, pz)):
            x0, _ = xweights(pj)
            x1 = jnp.minimum(x0 + 1, RESO - 1)
            r0 = j * (NUM_FRAMES * RESO) + rowbase + x0
            r1 = j * (NUM_FRAMES * RESO) + rowbase + x1
            pltpu.async_copy(tbl.at[r0], dbuf.at[slot3 * 6 + 2 * j], sem_d)
            pltpu.async_copy(tbl.at[r1], dbuf.at[slot3 * 6 + 2 * j + 1], sem_d)

    def stage_b(k, slot3, slot2):
        for j6 in range(6):
            pltpu.make_async_copy(tbl.at[pl.ds(0, CHUNK)],
                                  dbuf.at[slot3 * 6 + j6], sem_d).wait()
        px, py, pz, pt = chunk_cols(k)
        wxs = [xweights(pj)[1] for pj in (px, py, pz)]

        dbase = slot3 * 6

        def ch_body(c, accs):
            new = []
            for i in range(3):
                chn = _splat_i32(i * FEAT_F) + c
                prod = onef
                for j in range(3):
                    v0 = plsc.load_gather(dbuf, [_splat_i32(dbase + 2 * j),
                                                 lane, chn])
                    v1 = plsc.load_gather(dbuf, [_splat_i32(dbase + 2 * j + 1),
                                                 lane, chn])
                    prod = prod * (v0 + wxs[j] * (v1 - v0))
                new.append(accs[i] + prod)
            return tuple(new)

        zero = jnp.zeros((16,), dtype=jnp.float32)
        d0, d1, d2 = lax.fori_loop(0, FEAT_F, ch_body, (zero, zero, zero))

        hx = jnp.clip(px + d0, 0.0, np.float32(1.0 - EPS))
        hy = jnp.clip(py + d1, 0.0, np.float32(1.0 - EPS))
        hz = jnp.clip(pz + d2, 0.0, np.float32(1.0 - EPS))
        posq[0, pl.ds(slot2 * CHUNK, CHUNK)] = hx
        posq[1, pl.ds(slot2 * CHUNK, CHUNK)] = hy
        posq[2, pl.ds(slot2 * CHUNK, CHUNK)] = hz

        gbase = slot2 * NUM_LEVELS
        for l in range(NUM_LEVELS):
            sc = np.float32(SCALES[l])
            fx, fy, fz = hx * sc, hy * sc, hz * sc
            ix = fx.astype(jnp.int32)
            iy = fy.astype(jnp.int32)
            iz = fz.astype(jnp.int32)
            off = OFFSETS[l]

            def put(idx, l, c):
                idxbuf[gbase + l, 0, pl.ds(c * 16, 16)] = idx
                idxbuf[gbase + l, 1, pl.ds(c * 16, 16)] = idx + TOTAL_ENTRIES

            if l < START_HASH:
                sp = SCALES[l] + 1
                b0 = ix * (sp * sp) + iy * sp + iz + off
                for c in range(8):
                    bx, by, bz = c >> 2, (c >> 1) & 1, c & 1
                    put(b0 + (bx * sp * sp + by * sp + bz), l, c)
            else:
                h2, l2, h3, l3 = [], [], [], []
                for b in (iy, iy + 1):
                    lo = b * C2_LO
                    h2.append(b * C2_HI + (lo >> 16))
                    l2.append(jnp.bitwise_and(lo, 0xFFFF))
                for b in (iz, iz + 1):
                    lo = b * C3_LO
                    h3.append(b * C3_HI + (lo >> 16))
                    l3.append(jnp.bitwise_and(lo, 0xFFFF))
                wmod = {}
                for by in range(2):
                    for bz in range(2):
                        hh = jnp.bitwise_xor(h2[by], h3[bz])
                        wmod[(by, bz)] = _mod_p(_mod_p(hh) * 256)
                for c in range(8):
                    bx, by, bz = c >> 2, (c >> 1) & 1, c & 1
                    ll = jnp.bitwise_xor(jnp.bitwise_xor(ix + bx, l2[by]),
                                         l3[bz])
                    put(_mod_p(wmod[(by, bz)] * 256 + ll) + off, l, c)
            pltpu.async_copy(data.at[idxbuf.at[gbase + l, 0]],
                             ghash.at[gbase + l, 0], sem_h)
            pltpu.async_copy(data.at[idxbuf.at[gbase + l, 1]],
                             ghash.at[gbase + l, 1], sem_h)

    def stage_c(k, slot2):
        for w in range(2 * NUM_LEVELS):
            pltpu.make_async_copy(data.at[pl.ds(0, 8 * CHUNK)],
                                  ghash.at[slot2 * NUM_LEVELS + w // 2,
                                           w % 2], sem_h).wait()
        hx = posq[0, pl.ds(slot2 * CHUNK, CHUNK)]
        hy = posq[1, pl.ds(slot2 * CHUNK, CHUNK)]
        hz = posq[2, pl.ds(slot2 * CHUNK, CHUNK)]
        gbase = slot2 * NUM_LEVELS
        for l in range(NUM_LEVELS):
            sc = np.float32(SCALES[l])
            fx, fy, fz = hx * sc, hy * sc, hz * sc
            ox = fx - fx.astype(jnp.int32).astype(jnp.float32)
            oy = fy - fy.astype(jnp.int32).astype(jnp.float32)
            oz = fz - fz.astype(jnp.int32).astype(jnp.float32)
            ax = (onef - ox, ox)
            ay = (onef - oy, oy)
            az = (onef - oz, oz)
            for chn in range(LEVEL_DIM):
                val = jnp.zeros((16,), dtype=jnp.float32)
                for c in range(8):
                    gv = plsc.load_gather(
                        ghash, [_splat_i32(gbase + l), _splat_i32(chn),
                                _splat_i32(c * 16) + lane])
                    w8 = ax[c >> 2] * ay[(c >> 1) & 1] * az[c & 1]
                    val = val + w8 * gv
                plsc.store_scatter(outc, [_splat_i32(slot2 * CHUNK) + lane,
                                          _splat_i32(l * LEVEL_DIM + chn)],
                                   val)
        pltpu.async_copy(outc.at[pl.ds(slot2 * CHUNK, CHUNK)],
                         out.at[pl.ds(base_pt + k * CHUNK, CHUNK)], sem_o)

    fire_delta(0, 0)
    fire_delta(1, 1)

    def body(i, _):
        s3 = lax.rem(i, 3)
        s2 = jnp.bitwise_and(i, 1)

        @pl.when(i + 2 < NCHUNK)
        def _fire():
            fire_delta(i + 2, lax.rem(i + 2, 3))

        @pl.when(i < NCHUNK)
        def _b():
            stage_b(i, s3, s2)

        @pl.when(i >= 1)
        def _c():
            km = i - 1
            stage_c(km, jnp.bitwise_and(km, 1))

            @pl.when(km >= 1)
            def _drain_out():
                pltpu.make_async_copy(
                    out.at[pl.ds(0, CHUNK)],
                    outc.at[pl.ds(jnp.bitwise_and(km - 1, 1) * CHUNK, CHUNK)],
                    sem_o).wait()
        return _

    lax.fori_loop(0, NCHUNK + 1, body, None)
    pltpu.make_async_copy(
        out.at[pl.ds(0, CHUNK)],
        outc.at[pl.ds(((NCHUNK - 1) % 2) * CHUNK, CHUNK)], sem_o).wait()


@jax.jit
def _run(tbl, data, posw):
    mesh = plsc.VectorSubcoreMesh(core_axis_name="c", subcore_axis_name="s")
    return pl.kernel(
        _sc_body,
        out_type=jax.ShapeDtypeStruct((N_POINTS, NUM_LEVELS * LEVEL_DIM),
                                      jnp.float32),
        mesh=mesh,
        compiler_params=pltpu.CompilerParams(use_tc_tiling_on_sc=False,
                                             needs_layout_passes=False),
        scratch_types=[
            pltpu.VMEM((4, PW), jnp.float32),
            pltpu.VMEM((18, CHUNK, CH), jnp.float32),
            pltpu.VMEM((2 * NUM_LEVELS, LEVEL_DIM, 8 * CHUNK), jnp.int32),
            pltpu.VMEM((2 * NUM_LEVELS, LEVEL_DIM, 8 * CHUNK), jnp.float32),
            pltpu.VMEM((3, 2 * CHUNK), jnp.float32),
            pltpu.VMEM((2 * CHUNK, NUM_LEVELS * LEVEL_DIM), jnp.float32),
            pltpu.SemaphoreType.DMA,
            pltpu.SemaphoreType.DMA,
            pltpu.SemaphoreType.DMA,
        ],
    )(tbl, data, posw)


def kernel(x, wbounds, data, feat0, feat1, feat2):
    with jax.enable_x64(False):
        return _kernel32(x, wbounds, data, feat0, feat1, feat2)


_TF = 10


def _tbl_body(f0, f1, f2, out):
    for tt in range(_TF):
        cols = [f[0, :, pl.ds(tt * RESO, RESO)].T for f in (f0, f1, f2)]
        out[0, 0, tt] = jnp.concatenate(cols, axis=1)


def _build_tbl(feat0, feat1, feat2):
    nt = NUM_FRAMES // _TF
    spec = pl.BlockSpec((1, FEAT_F, _TF * RESO), lambda j, t: (j, 0, t))
    out = pl.pallas_call(
        _tbl_body,
        grid=(3, nt),
        in_specs=[spec, spec, spec],
        out_specs=pl.BlockSpec((1, 1, _TF, RESO, CH),
                               lambda j, t: (j, t, 0, 0, 0)),
        out_shape=jax.ShapeDtypeStruct((3, nt, _TF, RESO, CH), jnp.float32),
    )(feat0.reshape(3, FEAT_F, NUM_FRAMES * RESO),
      feat1.reshape(3, FEAT_F, NUM_FRAMES * RESO),
      feat2.reshape(3, FEAT_F, NUM_FRAMES * RESO))
    return out.reshape(TBL_ROWS, CH)


def _kernel32(x, wbounds, data, feat0, feat1, feat2):
    xyz = x[:, :3].astype(jnp.float32)
    t = x[:, 3].astype(jnp.float32)
    lo = wbounds[:3].astype(jnp.float32)
    hi = wbounds[3:6].astype(jnp.float32)
    inp = (jnp.clip(xyz, lo, hi) - lo) / jnp.max(hi - lo)
    posw = jnp.concatenate([inp, t[:, None]], axis=1)
    posw = posw.T.reshape(4, NW, PW).transpose(1, 0, 2)
    tbl = _build_tbl(feat0.astype(jnp.float32), feat1.astype(jnp.float32),
                     feat2.astype(jnp.float32))
    data_flat = jnp.transpose(data.astype(jnp.float32)).ravel()
    return _run(tbl, data_flat, posw.astype(jnp.float32))


# speedup vs baseline: 17.6998x; 1.0181x over previous
import functools

import jax
import jax.numpy as jnp
import numpy as np
from jax import lax
from jax.experimental import pallas as pl
from jax.experimental.pallas import tpu as pltpu
from jax.experimental.pallas import tpu_sc as plsc

EPS = 1e-06
NUM_LEVELS = 16
BASE_RESOLUTION = 16
DESIRED_RESOLUTION = 2048
LEVEL_DIM = 2
NUM_FRAMES = 100
FEAT_F = 64
RESO = 256
N_POINTS = 65536

_PRIME = 2 ** 19
def _isprime(n):
    i = 2
    while i * i <= n:
        if n % i == 0:
            return False
        i += 1
    return True
while not _isprime(_PRIME):
    _PRIME += 1
PRIME = _PRIME

B_SCALE = (DESIRED_RESOLUTION / BASE_RESOLUTION) ** (1.0 / (NUM_LEVELS - 1))
SCALES = []
OFFSETS = [0]
START_HASH = -1
for _i in range(NUM_LEVELS):
    _res = int(BASE_RESOLUTION * B_SCALE ** _i)
    SCALES.append(_res)
    _n_e = (_res + 1) ** 3
    if _n_e > PRIME:
        if START_HASH < 0:
            START_HASH = _i
        _n_e = PRIME
    OFFSETS.append(OFFSETS[-1] + _n_e)
TOTAL_ENTRIES = OFFSETS[-1]

C2_HI, C2_LO = 19349663 >> 16, 19349663 & 0xFFFF
C3_HI, C3_LO = 83492791 >> 16, 83492791 & 0xFFFF
INV_P = np.float32(1.0 / PRIME)

NW = 32
PW = N_POINTS // NW
CHUNK = 16
NCHUNK = PW // CHUNK
CH = 3 * FEAT_F
TBL_ROWS = 3 * NUM_FRAMES * RESO


def _splat_i32(v):
    return jnp.full((16,), v, dtype=jnp.int32)


def _mod_p(v):
    q = (v.astype(jnp.float32) * INV_P).astype(jnp.int32)
    r = v - q * PRIME
    r = jnp.where(r < 0, r + PRIME, r)
    r = jnp.where(r >= PRIME, r - PRIME, r)
    return r


def _sc_body(tbl, data, posw, out, pos_v, dbuf, idxbuf, ghash, posq, outc,
             sem_d, sem_h, sem_o):
    nc = 2
    wid = lax.axis_index("s") * nc + lax.axis_index("c")
    base_pt = wid * PW

    pltpu.sync_copy(posw.at[wid], pos_v)

    lane = lax.iota(jnp.int32, 16)
    onef = jnp.full((16,), 1.0, dtype=jnp.float32)

    def chunk_cols(k):
        s = k * CHUNK
        return [pos_v[d, pl.ds(s, CHUNK)] for d in range(4)]

    def xweights(pj):
        g = 2.0 * pj - 1.0
        gx = ((g + 1.0) * 0.5) * np.float32(RESO - 1)
        x0 = gx.astype(jnp.int32)
        return x0, gx - x0.astype(jnp.float32)

    def fire_delta(k, slot3):
        px, py, pz, pt = chunk_cols(k)
        rowbase = pt.astype(jnp.int32) * RESO
        for j, pj in enumerate((px, py, pz)):
            x0, _ = xweights(pj)
            x1 = jnp.minimum(x0 + 1, RESO - 1)
            r0 = j * (NUM_FRAMES * RESO) + rowbase + x0
            r1 = j * (NUM_FRAMES * RESO) + rowbase + x1
            pltpu.async_copy(tbl.at[r0], dbuf.at[slot3 * 6 + 2 * j], sem_d)
            pltpu.async_copy(tbl.at[r1], dbuf.at[slot3 * 6 + 2 * j + 1], sem_d)

    def stage_b(k, slot3, slot2):
        for j6 in range(6):
            pltpu.make_async_copy(tbl.at[pl.ds(0, CHUNK)],
                                  dbuf.at[slot3 * 6 + j6], sem_d).wait()
        px, py, pz, pt = chunk_cols(k)
        wxs = [xweights(pj)[1] for pj in (px, py, pz)]

        dbase = slot3 * 6

        def ch_body(c, accs):
            new = []
            for i in range(3):
                chn = _splat_i32(i * FEAT_F) + c
                prod = onef
                for j in range(3):
                    v0 = plsc.load_gather(dbuf, [_splat_i32(dbase + 2 * j),
                                                 lane, chn])
                    v1 = plsc.load_gather(dbuf, [_splat_i32(dbase + 2 * j + 1),
                                                 lane, chn])
                    prod = prod * (v0 + wxs[j] * (v1 - v0))
                new.append(accs[i] + prod)
            return tuple(new)

        zero = jnp.zeros((16,), dtype=jnp.float32)
        d0, d1, d2 = lax.fori_loop(0, FEAT_F, ch_body, (zero, zero, zero),
                                   unroll=4)

        hx = jnp.clip(px + d0, 0.0, np.float32(1.0 - EPS))
        hy = jnp.clip(py + d1, 0.0, np.float32(1.0 - EPS))
        hz = jnp.clip(pz + d2, 0.0, np.float32(1.0 - EPS))
        posq[0, pl.ds(slot2 * CHUNK, CHUNK)] = hx
        posq[1, pl.ds(slot2 * CHUNK, CHUNK)] = hy
        posq[2, pl.ds(slot2 * CHUNK, CHUNK)] = hz

        gbase = slot2 * NUM_LEVELS
        for l in range(NUM_LEVELS):
            sc = np.float32(SCALES[l])
            fx, fy, fz = hx * sc, hy * sc, hz * sc
            ix = fx.astype(jnp.int32)
            iy = fy.astype(jnp.int32)
            iz = fz.astype(jnp.int32)
            off = OFFSETS[l]

            def put(idx, l, c):
                idxbuf[slot2, 0, pl.ds(l * 128 + c * 16, 16)] = idx
                idxbuf[slot2, 1, pl.ds(l * 128 + c * 16, 16)] = (
                    idx + TOTAL_ENTRIES)

            if l < START_HASH:
                sp = SCALES[l] + 1
                b0 = ix * (sp * sp) + iy * sp + iz + off
                for c in range(8):
                    bx, by, bz = c >> 2, (c >> 1) & 1, c & 1
                    put(b0 + (bx * sp * sp + by * sp + bz), l, c)
            else:
                h2, l2, h3, l3 = [], [], [], []
                for b in (iy, iy + 1):
                    lo = b * C2_LO
                    h2.append(b * C2_HI + (lo >> 16))
                    l2.append(jnp.bitwise_and(lo, 0xFFFF))
                for b in (iz, iz + 1):
                    lo = b * C3_LO
                    h3.append(b * C3_HI + (lo >> 16))
                    l3.append(jnp.bitwise_and(lo, 0xFFFF))
                wmod = {}
                for by in range(2):
                    for bz in range(2):
                        hh = jnp.bitwise_xor(h2[by], h3[bz])
                        wmod[(by, bz)] = _mod_p(_mod_p(hh) * 256)
                for c in range(8):
                    bx, by, bz = c >> 2, (c >> 1) & 1, c & 1
                    ll = jnp.bitwise_xor(jnp.bitwise_xor(ix + bx, l2[by]),
                                         l3[bz])
                    put(_mod_p(wmod[(by, bz)] * 256 + ll) + off, l, c)
        pltpu.async_copy(data.at[idxbuf.at[slot2, 0]],
                         ghash.at[slot2, 0], sem_h)
        pltpu.async_copy(data.at[idxbuf.at[slot2, 1]],
                         ghash.at[slot2, 1], sem_h)

    def stage_c(k, slot2):
        for chd in range(2):
            pltpu.make_async_copy(data.at[pl.ds(0, NUM_LEVELS * 8 * CHUNK)],
                                  ghash.at[slot2, chd], sem_h).wait()
        hx = posq[0, pl.ds(slot2 * CHUNK, CHUNK)]
        hy = posq[1, pl.ds(slot2 * CHUNK, CHUNK)]
        hz = posq[2, pl.ds(slot2 * CHUNK, CHUNK)]
        gbase = slot2 * NUM_LEVELS
        for l in range(NUM_LEVELS):
            sc = np.float32(SCALES[l])
            fx, fy, fz = hx * sc, hy * sc, hz * sc
            ox = fx - fx.astype(jnp.int32).astype(jnp.float32)
            oy = fy - fy.astype(jnp.int32).astype(jnp.float32)
            oz = fz - fz.astype(jnp.int32).astype(jnp.float32)
            ax = (onef - ox, ox)
            ay = (onef - oy, oy)
            az = (onef - oz, oz)
            for chn in range(LEVEL_DIM):
                val = jnp.zeros((16,), dtype=jnp.float32)
                for c in range(8):
                    gv = plsc.load_gather(
                        ghash, [_splat_i32(slot2), _splat_i32(chn),
                                _splat_i32(l * 128 + c * 16) + lane])
                    w8 = ax[c >> 2] * ay[(c >> 1) & 1] * az[c & 1]
                    val = val + w8 * gv
                plsc.store_scatter(outc, [_splat_i32(slot2 * CHUNK) + lane,
                                          _splat_i32(l * LEVEL_DIM + chn)],
                                   val)
        pltpu.async_copy(outc.at[pl.ds(slot2 * CHUNK, CHUNK)],
                         out.at[pl.ds(base_pt + k * CHUNK, CHUNK)], sem_o)

    fire_delta(0, 0)
    fire_delta(1, 1)

    def body(i, _):
        s3 = lax.rem(i, 3)
        s2 = jnp.bitwise_and(i, 1)

        @pl.when(i + 2 < NCHUNK)
        def _fire():
            fire_delta(i + 2, lax.rem(i + 2, 3))

        @pl.when(i < NCHUNK)
        def _b():
            stage_b(i, s3, s2)

        @pl.when(i >= 1)
        def _c():
            km = i - 1
            stage_c(km, jnp.bitwise_and(km, 1))

            @pl.when(km >= 1)
            def _drain_out():
                pltpu.make_async_copy(
                    out.at[pl.ds(0, CHUNK)],
                    outc.at[pl.ds(jnp.bitwise_and(km - 1, 1) * CHUNK, CHUNK)],
                    sem_o).wait()
        return _

    lax.fori_loop(0, NCHUNK + 1, body, None)
    pltpu.make_async_copy(
        out.at[pl.ds(0, CHUNK)],
        outc.at[pl.ds(((NCHUNK - 1) % 2) * CHUNK, CHUNK)], sem_o).wait()


@jax.jit
def _run(tbl, data, posw):
    mesh = plsc.VectorSubcoreMesh(core_axis_name="c", subcore_axis_name="s")
    return pl.kernel(
        _sc_body,
        out_type=jax.ShapeDtypeStruct((N_POINTS, NUM_LEVELS * LEVEL_DIM),
                                      jnp.float32),
        mesh=mesh,
        compiler_params=pltpu.CompilerParams(use_tc_tiling_on_sc=False,
                                             needs_layout_passes=False),
        scratch_types=[
            pltpu.VMEM((4, PW), jnp.float32),
            pltpu.VMEM((18, CHUNK, CH), jnp.float32),
            pltpu.VMEM((2, LEVEL_DIM, NUM_LEVELS * 8 * CHUNK), jnp.int32),
            pltpu.VMEM((2, LEVEL_DIM, NUM_LEVELS * 8 * CHUNK), jnp.float32),
            pltpu.VMEM((3, 2 * CHUNK), jnp.float32),
            pltpu.VMEM((2 * CHUNK, NUM_LEVELS * LEVEL_DIM), jnp.float32),
            pltpu.SemaphoreType.DMA,
            pltpu.SemaphoreType.DMA,
            pltpu.SemaphoreType.DMA,
        ],
    )(tbl, data, posw)


def kernel(x, wbounds, data, feat0, feat1, feat2):
    with jax.enable_x64(False):
        return _kernel32(x, wbounds, data, feat0, feat1, feat2)


_TF = 10


def _tbl_body(f0, f1, f2, out):
    for tt in range(_TF):
        cols = [f[0, :, pl.ds(tt * RESO, RESO)].T for f in (f0, f1, f2)]
        out[0, 0, tt] = jnp.concatenate(cols, axis=1)


def _build_tbl(feat0, feat1, feat2):
    nt = NUM_FRAMES // _TF
    spec = pl.BlockSpec((1, FEAT_F, _TF * RESO), lambda j, t: (j, 0, t))
    out = pl.pallas_call(
        _tbl_body,
        grid=(3, nt),
        in_specs=[spec, spec, spec],
        out_specs=pl.BlockSpec((1, 1, _TF, RESO, CH),
                               lambda j, t: (j, t, 0, 0, 0)),
        out_shape=jax.ShapeDtypeStruct((3, nt, _TF, RESO, CH), jnp.float32),
    )(feat0.reshape(3, FEAT_F, NUM_FRAMES * RESO),
      feat1.reshape(3, FEAT_F, NUM_FRAMES * RESO),
      feat2.reshape(3, FEAT_F, NUM_FRAMES * RESO))
    return out.reshape(TBL_ROWS, CH)


def _kernel32(x, wbounds, data, feat0, feat1, feat2):
    xyz = x[:, :3].astype(jnp.float32)
    t = x[:, 3].astype(jnp.float32)
    lo = wbounds[:3].astype(jnp.float32)
    hi = wbounds[3:6].astype(jnp.float32)
    inp = (jnp.clip(xyz, lo, hi) - lo) / jnp.max(hi - lo)
    posw = jnp.concatenate([inp, t[:, None]], axis=1)
    posw = posw.T.reshape(4, NW, PW).transpose(1, 0, 2)
    tbl = _build_tbl(feat0.astype(jnp.float32), feat1.astype(jnp.float32),
                     feat2.astype(jnp.float32))
    data_flat = jnp.transpose(data.astype(jnp.float32)).ravel()
    return _run(tbl, data_flat, posw.astype(jnp.float32))
